# baseline (speedup 1.0000x reference)
import functools

import jax
import jax.numpy as jnp
from jax import lax
from jax.experimental import pallas as pl
from jax.experimental.pallas import tpu as pltpu
from jax.experimental.pallas import tpu_sc as plsc

NUM_CORES = 2
NUM_SUBCORES = 16
NUM_WORKERS = NUM_CORES * NUM_SUBCORES
GRP = 8
PITCH = 40


@functools.cache
def _make_kernel(batch, seq_len, vocab, embed):
  assert batch == 128 * NUM_WORKERS and embed == 32 and seq_len % GRP == 0
  n_grp = seq_len // GRP
  grp_rows = GRP * 128
  mesh = plsc.VectorSubcoreMesh(
      core_axis_name="c", subcore_axis_name="s",
      num_cores=NUM_CORES, num_subcores=NUM_SUBCORES)

  @functools.partial(
      pl.kernel,
      out_type=jax.ShapeDtypeStruct(
          (seq_len, embed // 8, batch // 128, 8, 128), jnp.float32),
      mesh=mesh,
      compiler_params=pltpu.CompilerParams(
          use_tc_tiling_on_sc=False, needs_layout_passes=False),
      scratch_types=[
          pltpu.VMEM((grp_rows,), jnp.int32),
          pltpu.VMEM((grp_rows, embed), jnp.float32),
          pltpu.VMEM((grp_rows, PITCH), jnp.float32),
          pltpu.VMEM((2, embed // 8, 8, 128), jnp.float32),
          pltpu.VMEM((seq_len, embed), jnp.float32),
          pltpu.VMEM_SHARED((NUM_SUBCORES, grp_rows, embed), jnp.float32),
          pltpu.SemaphoreType.DMA,
          pltpu.SemaphoreType.DMA((2,)),
      ],
  )
  def k(table_hbm, idx_hbm, pos_hbm, out_hbm, idx_v, g_tmp, g_v, t_v, pos_v,
        sp_sh, gsem, wsem):
    s_ax = lax.axis_index("s")
    wid = s_ax * NUM_CORES + lax.axis_index("c")
    pltpu.sync_copy(pos_hbm, pos_v)
    iota = lax.iota(jnp.int32, 16)

    def drain_write(tb):
      pltpu.make_async_copy(t_v.at[tb], out_hbm.at[0, :, wid, :, :],
                            wsem.at[tb]).wait()

    def body(g, _):
      for j in range(GRP):
        src_off = pl.multiple_of((g * GRP + j) * batch + wid * 128, 128)
        pltpu.sync_copy(idx_hbm.at[pl.ds(src_off, 128)],
                        idx_v.at[pl.ds(j * 128, 128)])
      pltpu.async_copy(table_hbm.at[idx_v], g_tmp, gsem).wait()
      pltpu.sync_copy(g_tmp, sp_sh.at[s_ax])
      pltpu.sync_copy(sp_sh.at[s_ax], g_v.at[:, pl.ds(0, embed)])

      for j in range(GRP):
        l = g * GRP + j
        tb = j % 2
        if j >= 2:
          drain_write(tb)
        else:
          @pl.when(g > 0)
          def _():
            drain_write(tb)
        bl = jnp.broadcast_to(l, (16,)).astype(jnp.int32)
        base = jnp.broadcast_to(j * 128, (16,)).astype(jnp.int32)
        rows_q = [base + (16 * q + iota) for q in range(8)]

        @plsc.parallel_loop(0, embed // 8)
        def _(r):
          for s in range(8):
            bd = jnp.broadcast_to(r * 8 + s, (16,)).astype(jnp.int32)
            pd = plsc.load_gather(pos_v, [bl, bd])
            for q in range(8):
              v = plsc.load_gather(g_v, [rows_q[q], bd])
              t_v[tb, r, s, pl.ds(16 * q, 16)] = v + pd
        pltpu.async_copy(t_v.at[tb], out_hbm.at[l, :, wid, :, :],
                         wsem.at[tb])
      return ()

    lax.fori_loop(0, n_grp, body, (), unroll=False)
    drain_write(0)
    drain_write(1)

  return k


def kernel(inputs, token_table, pos_table):
  batch, seq_len = inputs.shape
  vocab, embed = token_table.shape
  idx = inputs.transpose(1, 0).reshape(batch * seq_len).astype(jnp.int32)
  k = _make_kernel(batch, seq_len, vocab, embed)
  o5 = k(token_table, idx, pos_table)
  return o5.transpose(2, 4, 0, 1, 3).reshape(batch, seq_len, embed)


# speedup vs baseline: 1.1009x; 1.1009x over previous
import functools

import jax
import jax.numpy as jnp
from jax import lax
from jax.experimental import pallas as pl
from jax.experimental.pallas import tpu as pltpu
from jax.experimental.pallas import tpu_sc as plsc

NUM_CORES = 2
NUM_SUBCORES = 16
NUM_WORKERS = NUM_CORES * NUM_SUBCORES


@functools.cache
def _make_kernel(n_rows, seq_len, embed, chunk_rows):
  assert n_rows % (NUM_WORKERS * chunk_rows) == 0
  assert chunk_rows % seq_len == 0
  rows_per_worker = n_rows // NUM_WORKERS
  n_chunks = rows_per_worker // chunk_rows
  seqs_per_chunk = chunk_rows // seq_len
  assert n_chunks >= 2

  mesh = plsc.VectorSubcoreMesh(
      core_axis_name="c", subcore_axis_name="s",
      num_cores=NUM_CORES, num_subcores=NUM_SUBCORES)

  @functools.partial(
      pl.kernel,
      out_type=jax.ShapeDtypeStruct((n_rows, embed), jnp.float32),
      mesh=mesh,
      compiler_params=pltpu.CompilerParams(use_tc_tiling_on_sc=False),
      scratch_types=[
          pltpu.VMEM((2 * chunk_rows,), jnp.int32),
          pltpu.VMEM((2 * chunk_rows, embed), jnp.float32),
          pltpu.VMEM_SHARED((chunk_rows, embed), jnp.float32),
          pltpu.SemaphoreType.DMA((2,)),
          pltpu.SemaphoreType.DMA((2,)),
      ],
  )
  def k(table_hbm, idx_hbm, pos_hbm, out_hbm, idx_v, rows_v, pos_sh,
        gsem, wsem):
    sid = lax.axis_index("s")
    wid = sid * NUM_CORES + lax.axis_index("c")
    base = wid * rows_per_worker

    @pl.when(sid == 0)
    def _():
      for s in range(seqs_per_chunk):
        pltpu.sync_copy(pos_hbm,
                        pos_sh.at[pl.ds(s * seq_len, seq_len)])

    plsc.subcore_barrier()

    def load_idx(c, buf):
      src = pl.multiple_of(base + c * chunk_rows, chunk_rows)
      dst = pl.multiple_of(buf * chunk_rows, chunk_rows)
      pltpu.sync_copy(idx_hbm.at[pl.ds(src, chunk_rows)],
                      idx_v.at[pl.ds(dst, chunk_rows)])

    def prefill(buf):
      dst = pl.multiple_of(buf * chunk_rows, chunk_rows)
      pltpu.sync_copy(pos_sh, rows_v.at[pl.ds(dst, chunk_rows)])

    def fire_gather(buf):
      off = pl.multiple_of(buf * chunk_rows, chunk_rows)
      pltpu.async_copy(table_hbm.at[idx_v.at[pl.ds(off, chunk_rows)]],
                       rows_v.at[pl.ds(off, chunk_rows)], gsem.at[buf],
                       add=True)

    def drain_gather(buf):
      off = pl.multiple_of(buf * chunk_rows, chunk_rows)
      pltpu.make_async_copy(table_hbm.at[pl.ds(0, chunk_rows)],
                            rows_v.at[pl.ds(off, chunk_rows)],
                            gsem.at[buf]).wait()

    def fire_write(c, buf):
      src = pl.multiple_of(buf * chunk_rows, chunk_rows)
      dst = pl.multiple_of(base + c * chunk_rows, chunk_rows)
      pltpu.async_copy(rows_v.at[pl.ds(src, chunk_rows)],
                       out_hbm.at[pl.ds(dst, chunk_rows)], wsem.at[buf])

    def drain_write(buf):
      src = pl.multiple_of(buf * chunk_rows, chunk_rows)
      pltpu.make_async_copy(rows_v.at[pl.ds(src, chunk_rows)],
                            out_hbm.at[pl.ds(0, chunk_rows)],
                            wsem.at[buf]).wait()

    load_idx(0, 0)
    prefill(0)
    fire_gather(0)
    load_idx(1, 1)

    def body(c, _):
      buf = lax.rem(c, 2)
      nxt = 1 - buf
      drain_gather(buf)
      fire_write(c, buf)

      @pl.when(c + 2 < n_chunks)
      def _():
        load_idx(c + 2, buf)

      @pl.when(c + 1 < n_chunks)
      def _():
        @pl.when(c > 0)
        def _():
          drain_write(nxt)
        prefill(nxt)
        fire_gather(nxt)

      return ()

    lax.fori_loop(0, n_chunks, body, (), unroll=False)
    drain_write(0)
    drain_write(1)

  return k


def kernel(inputs, token_table, pos_table):
  batch, seq_len = inputs.shape
  _, embed = token_table.shape
  n_rows = batch * seq_len
  idx = inputs.reshape(n_rows).astype(jnp.int32)
  k = _make_kernel(n_rows, seq_len, embed, chunk_rows=1600)
  out = k(token_table, idx, pos_table)
  return out.reshape(batch, seq_len, embed)
